# Initial kernel scaffold; baseline (speedup 1.0000x reference)
import jax
import jax.numpy as jnp
from jax.experimental import pallas as pl


def kernel(edges_data, nodes_data, global_data, receivers, senders):
    raise NotImplementedError("write your pallas kernel here")


# speedup vs baseline: 1.3371x; 1.3371x over previous
import functools

import jax
import jax.numpy as jnp
from jax import lax
from jax.experimental import pallas as pl
from jax.experimental.pallas import tpu as pltpu
from jax.experimental.pallas import tpu_sc as plsc

N_NODES = 10000
N_EDGES = 320000
D_FEAT = 128
D_EDGE = 16
D_GLOBAL = 128
D_OUT = D_EDGE + 2 * D_FEAT + D_GLOBAL

NUM_CORES = 2
NUM_SUBCORES = 16
NW = NUM_CORES * NUM_SUBCORES
E_PER_W = N_EDGES // NW
CHUNK = 80
N_CHUNKS = E_PER_W // CHUNK

_mesh = plsc.VectorSubcoreMesh(core_axis_name="c", subcore_axis_name="s")


@functools.partial(
    pl.kernel,
    out_type=jax.ShapeDtypeStruct((N_EDGES, D_OUT), jnp.float32),
    mesh=_mesh,
    compiler_params=pltpu.CompilerParams(use_tc_tiling_on_sc=False),
    scratch_types=[
        pltpu.VMEM((CHUNK,), jnp.int32),
        pltpu.VMEM((CHUNK,), jnp.int32),
        pltpu.VMEM((CHUNK, D_FEAT), jnp.float32),
        pltpu.VMEM((CHUNK, D_FEAT), jnp.float32),
        pltpu.VMEM((CHUNK, D_EDGE), jnp.float32),
        pltpu.VMEM((CHUNK, D_GLOBAL), jnp.float32),
        pltpu.SemaphoreType.DMA,
        pltpu.SemaphoreType.DMA,
        pltpu.SemaphoreType.DMA,
        pltpu.SemaphoreType.DMA,
    ],
)
def _edge_block_sc(
    edges_hbm, nodes_hbm, glob_hbm, recv_hbm, send_hbm, out_hbm,
    idx_r, idx_s, rows_r, rows_s, edg_v, glob_v,
    sem_r, sem_s, sem_e, sem_g,
):
    wid = lax.axis_index("s") * NUM_CORES + lax.axis_index("c")
    wbase = wid * E_PER_W

    pltpu.sync_copy(glob_hbm, glob_v.at[0])
    for j in range(D_GLOBAL // 16):
        row = glob_v[0, pl.ds(j * 16, 16)]
        for i in range(1, CHUNK):
            glob_v[i, pl.ds(j * 16, 16)] = row

    def chunk_body(c, carry):
        base = wbase + c * CHUNK
        pltpu.sync_copy(recv_hbm.at[pl.ds(base, CHUNK)], idx_r)
        pltpu.sync_copy(send_hbm.at[pl.ds(base, CHUNK)], idx_s)
        g_r = pltpu.async_copy(nodes_hbm.at[idx_r], rows_r, sem_r)
        g_s = pltpu.async_copy(nodes_hbm.at[idx_s], rows_s, sem_s)
        g_e = pltpu.async_copy(edges_hbm.at[pl.ds(base, CHUNK)], edg_v, sem_e)
        w_g = pltpu.async_copy(
            glob_v, out_hbm.at[pl.ds(base, CHUNK), pl.ds(D_EDGE + 2 * D_FEAT, D_GLOBAL)],
            sem_g)
        g_r.wait()
        pltpu.sync_copy(rows_r, out_hbm.at[pl.ds(base, CHUNK), pl.ds(D_EDGE, D_FEAT)])
        g_s.wait()
        pltpu.sync_copy(
            rows_s, out_hbm.at[pl.ds(base, CHUNK), pl.ds(D_EDGE + D_FEAT, D_FEAT)])
        g_e.wait()
        pltpu.sync_copy(edg_v, out_hbm.at[pl.ds(base, CHUNK), pl.ds(0, D_EDGE)])
        w_g.wait()
        return carry

    lax.fori_loop(0, N_CHUNKS, chunk_body, 0)


def kernel(edges_data, nodes_data, global_data, receivers, senders):
    return _edge_block_sc(
        edges_data,
        nodes_data,
        global_data,
        receivers.astype(jnp.int32),
        senders.astype(jnp.int32),
    )


# speedup vs baseline: 1.4557x; 1.0887x over previous
import functools

import jax
import jax.numpy as jnp
from jax import lax
from jax.experimental import pallas as pl
from jax.experimental.pallas import tpu as pltpu
from jax.experimental.pallas import tpu_sc as plsc

N_NODES = 10000
N_EDGES = 320000
D_FEAT = 128
D_EDGE = 16
D_GLOBAL = 128
D_OUT = D_EDGE + 2 * D_FEAT + D_GLOBAL
COL_R = D_EDGE
COL_S = D_EDGE + D_FEAT
COL_G = D_EDGE + 2 * D_FEAT

NUM_CORES = 2
NUM_SUBCORES = 16
NW = NUM_CORES * NUM_SUBCORES
E_PER_W = N_EDGES // NW
CHUNK = 80
N_CHUNKS = E_PER_W // CHUNK
NSLOT = 3
MAIN_STEPS = (N_CHUNKS - 2) // NSLOT
TAIL = N_CHUNKS - MAIN_STEPS * NSLOT

_mesh = plsc.VectorSubcoreMesh(core_axis_name="c", subcore_axis_name="s")


@functools.partial(
    pl.kernel,
    out_type=jax.ShapeDtypeStruct((N_EDGES, D_OUT), jnp.float32),
    mesh=_mesh,
    compiler_params=pltpu.CompilerParams(use_tc_tiling_on_sc=False),
    scratch_types=[
        pltpu.VMEM((E_PER_W,), jnp.int32),
        pltpu.VMEM((E_PER_W,), jnp.int32),
        [pltpu.VMEM((CHUNK, D_FEAT), jnp.float32)] * NSLOT,
        [pltpu.VMEM((CHUNK, D_FEAT), jnp.float32)] * NSLOT,
        [pltpu.VMEM((CHUNK, D_EDGE), jnp.float32)] * NSLOT,
        pltpu.VMEM((CHUNK, D_GLOBAL), jnp.float32),
        [pltpu.SemaphoreType.DMA] * NSLOT,
        [pltpu.SemaphoreType.DMA] * NSLOT,
    ],
)
def _edge_block_sc(
    edges_hbm, nodes_hbm, glob_hbm, recv_hbm, send_hbm, out_hbm,
    idx_r_all, idx_s_all, rows_r, rows_s, edg_v, glob_v, gsem, wsem,
):
    wid = lax.axis_index("s") * NUM_CORES + lax.axis_index("c")
    wbase = wid * E_PER_W

    pltpu.sync_copy(recv_hbm.at[pl.ds(wbase, E_PER_W)], idx_r_all)
    pltpu.sync_copy(send_hbm.at[pl.ds(wbase, E_PER_W)], idx_s_all)

    pltpu.sync_copy(glob_hbm, glob_v.at[0])
    for j in range(D_GLOBAL // 16):
        row = glob_v[0, pl.ds(j * 16, 16)]
        for i in range(1, CHUNK):
            glob_v[i, pl.ds(j * 16, 16)] = row

    def gather_descs(s, c):
        loc = c * CHUNK
        return [
            pltpu.make_async_copy(
                nodes_hbm.at[idx_r_all.at[pl.ds(loc, CHUNK)]], rows_r[s], gsem[s]),
            pltpu.make_async_copy(
                nodes_hbm.at[idx_s_all.at[pl.ds(loc, CHUNK)]], rows_s[s], gsem[s]),
            pltpu.make_async_copy(
                edges_hbm.at[pl.ds(wbase + loc, CHUNK)], edg_v[s], gsem[s]),
        ]

    def write_descs(s, c):
        base = wbase + c * CHUNK
        rows = pl.ds(base, CHUNK)
        return [
            pltpu.make_async_copy(rows_r[s], out_hbm.at[rows, pl.ds(COL_R, D_FEAT)], wsem[s]),
            pltpu.make_async_copy(rows_s[s], out_hbm.at[rows, pl.ds(COL_S, D_FEAT)], wsem[s]),
            pltpu.make_async_copy(edg_v[s], out_hbm.at[rows, pl.ds(0, D_EDGE)], wsem[s]),
            pltpu.make_async_copy(glob_v, out_hbm.at[rows, pl.ds(COL_G, D_GLOBAL)], wsem[s]),
        ]

    def pipe_step(s, c):
        sn = (s + 1) % NSLOT

        @pl.when(c >= 2)
        def _():
            for d in write_descs(sn, c):
                d.wait()

        @pl.when(c + 1 < N_CHUNKS)
        def _():
            for d in gather_descs(sn, c + 1):
                d.start()

        for d in gather_descs(s, c):
            d.wait()
        for d in write_descs(s, c):
            d.start()

    for d in gather_descs(0, jnp.int32(0)):
        d.start()

    def body(k, carry):
        c0 = k * NSLOT
        for j in range(NSLOT):
            pipe_step(j, c0 + j)
        return carry

    lax.fori_loop(0, MAIN_STEPS, body, 0)
    for t in range(TAIL):
        pipe_step(t, jnp.int32(MAIN_STEPS * NSLOT + t))
    for t in range(TAIL):
        for d in write_descs(t, jnp.int32(N_CHUNKS - TAIL + t)):
            d.wait()


def kernel(edges_data, nodes_data, global_data, receivers, senders):
    return _edge_block_sc(
        edges_data,
        nodes_data,
        global_data,
        receivers.astype(jnp.int32),
        senders.astype(jnp.int32),
    )
